# initial kernel scaffold (unmeasured)
import jax
import jax.numpy as jnp
from jax import lax
from jax.experimental import pallas as pl
from jax.experimental.pallas import tpu as pltpu

H, S, D = 16, 1024, 128
HH = H // 2
SCALE = D ** -0.5


def _body(q_ref, k_ref, v_ref, out_ref, k_rem, v_rem,
          send_x, recv_x, send_y, recv_y):
    my_x = lax.axis_index("x")
    my_y = lax.axis_index("y")

    barrier = pltpu.get_barrier_semaphore()
    for nbr in ((1 - my_x, my_y), (my_x, 1 - my_y)):
        pl.semaphore_signal(barrier, inc=1, device_id=nbr,
                            device_id_type=pl.DeviceIdType.MESH)
    pl.semaphore_wait(barrier, 2)

    hs = HH * my_y

    p1 = []
    for i, (src, dst) in enumerate(((k_ref, k_rem), (v_ref, v_rem))):
        rdma = pltpu.make_async_remote_copy(
            src_ref=src.at[pl.ds(hs, HH)],
            dst_ref=dst.at[pl.ds(hs, HH)],
            send_sem=send_x.at[i],
            recv_sem=recv_x.at[i],
            device_id=(1 - my_x, my_y),
            device_id_type=pl.DeviceIdType.MESH,
        )
        rdma.start()
        p1.append(rdma)
    for rdma in p1:
        rdma.wait()

    p2 = []
    for i, buf in enumerate((k_rem, v_rem)):
        rdma = pltpu.make_async_remote_copy(
            src_ref=buf.at[pl.ds(hs, HH)],
            dst_ref=buf.at[pl.ds(hs, HH)],
            send_sem=send_y.at[i],
            recv_sem=recv_y.at[i],
            device_id=(my_x, 1 - my_y),
            device_id_type=pl.DeviceIdType.MESH,
        )
        rdma.start()
        p2.append(rdma)
    for rdma in p2:
        rdma.wait()

    for h in range(H):
        q = q_ref[h]
        s_loc = lax.dot_general(q, k_ref[h], (((1,), (1,)), ((), ())),
                                preferred_element_type=jnp.float32) * SCALE
        s_rem = lax.dot_general(q, k_rem[h], (((1,), (1,)), ((), ())),
                                preferred_element_type=jnp.float32) * SCALE
        m = jnp.maximum(jnp.max(s_loc, axis=1, keepdims=True),
                        jnp.max(s_rem, axis=1, keepdims=True))
        p_loc = jnp.exp(s_loc - m)
        p_rem = jnp.exp(s_rem - m)
        denom = (jnp.sum(p_loc, axis=1, keepdims=True)
                 + jnp.sum(p_rem, axis=1, keepdims=True))
        acc = lax.dot_general(p_loc.astype(jnp.bfloat16), v_ref[h],
                              (((1,), (0,)), ((), ())),
                              preferred_element_type=jnp.float32)
        acc = acc + lax.dot_general(p_rem.astype(jnp.bfloat16), v_rem[h],
                                    (((1,), (0,)), ((), ())),
                                    preferred_element_type=jnp.float32)
        out_ref[h] = acc / denom


def kernel(Q, K, V):
    q = jnp.transpose(Q[0].astype(jnp.bfloat16), (1, 0, 2))
    k = jnp.transpose(K[0].astype(jnp.bfloat16), (1, 0, 2))
    v = jnp.transpose(V[0].astype(jnp.bfloat16), (1, 0, 2))
    out = pl.pallas_call(
        _body,
        out_shape=jax.ShapeDtypeStruct((H, S, D), jnp.float32),
        in_specs=[pl.BlockSpec(memory_space=pltpu.VMEM)] * 3,
        out_specs=pl.BlockSpec(memory_space=pltpu.VMEM),
        scratch_shapes=[
            pltpu.VMEM((H, S, D), jnp.bfloat16),
            pltpu.VMEM((H, S, D), jnp.bfloat16),
            pltpu.SemaphoreType.DMA((2,)),
            pltpu.SemaphoreType.DMA((2,)),
            pltpu.SemaphoreType.DMA((2,)),
            pltpu.SemaphoreType.DMA((2,)),
        ],
        compiler_params=pltpu.CompilerParams(collective_id=0),
    )(q, k, v)
    return jnp.transpose(out, (1, 0, 2))[None]


# baseline (device time: 211051 ns/iter reference)
import jax
import jax.numpy as jnp
from jax import lax
from jax.experimental import pallas as pl
from jax.experimental.pallas import tpu as pltpu

H, S, D = 16, 1024, 128
HH = H // 2
SCALE = D ** -0.5


def _body(q_ref, k_ref, v_ref, out_ref, k_rem, v_rem,
          send_x, recv_x, send_y, recv_y):
    my_x = lax.axis_index("x")
    my_y = lax.axis_index("y")

    barrier = pltpu.get_barrier_semaphore()
    for nbr in ((1 - my_x, my_y), (my_x, 1 - my_y)):
        pl.semaphore_signal(barrier, inc=1, device_id=nbr,
                            device_id_type=pl.DeviceIdType.MESH)
    pl.semaphore_wait(barrier, 2)

    hs = HH * my_y

    p1 = []
    for i, (src, dst) in enumerate(((k_ref, k_rem), (v_ref, v_rem))):
        rdma = pltpu.make_async_remote_copy(
            src_ref=src.at[pl.ds(hs, HH)],
            dst_ref=dst.at[pl.ds(hs, HH)],
            send_sem=send_x.at[i],
            recv_sem=recv_x.at[i],
            device_id=(1 - my_x, my_y),
            device_id_type=pl.DeviceIdType.MESH,
        )
        rdma.start()
        p1.append(rdma)
    for rdma in p1:
        rdma.wait()

    p2 = []
    for i, buf in enumerate((k_rem, v_rem)):
        rdma = pltpu.make_async_remote_copy(
            src_ref=buf.at[pl.ds(hs, HH)],
            dst_ref=buf.at[pl.ds(hs, HH)],
            send_sem=send_y.at[i],
            recv_sem=recv_y.at[i],
            device_id=(my_x, 1 - my_y),
            device_id_type=pl.DeviceIdType.MESH,
        )
        rdma.start()
        p2.append(rdma)
    for rdma in p2:
        rdma.wait()

    def head_step(h, carry):
        q = q_ref[h]
        s_loc = lax.dot_general(q, k_ref[h], (((1,), (1,)), ((), ())),
                                preferred_element_type=jnp.float32) * SCALE
        s_rem = lax.dot_general(q, k_rem[h], (((1,), (1,)), ((), ())),
                                preferred_element_type=jnp.float32) * SCALE
        m = jnp.maximum(jnp.max(s_loc, axis=1, keepdims=True),
                        jnp.max(s_rem, axis=1, keepdims=True))
        p_loc = jnp.exp(s_loc - m)
        p_rem = jnp.exp(s_rem - m)
        denom = (jnp.sum(p_loc, axis=1, keepdims=True)
                 + jnp.sum(p_rem, axis=1, keepdims=True))
        acc = lax.dot_general(p_loc.astype(jnp.bfloat16), v_ref[h],
                              (((1,), (0,)), ((), ())),
                              preferred_element_type=jnp.float32)
        acc = acc + lax.dot_general(p_rem.astype(jnp.bfloat16), v_rem[h],
                                    (((1,), (0,)), ((), ())),
                                    preferred_element_type=jnp.float32)
        out_ref[h] = acc / denom
        return carry

    lax.fori_loop(0, H, head_step, 0)


def kernel(Q, K, V):
    q = jnp.transpose(Q[0].astype(jnp.bfloat16), (1, 0, 2))
    k = jnp.transpose(K[0].astype(jnp.bfloat16), (1, 0, 2))
    v = jnp.transpose(V[0].astype(jnp.bfloat16), (1, 0, 2))
    out = pl.pallas_call(
        _body,
        out_shape=jax.ShapeDtypeStruct((H, S, D), jnp.float32),
        in_specs=[pl.BlockSpec(memory_space=pltpu.VMEM)] * 3,
        out_specs=pl.BlockSpec(memory_space=pltpu.VMEM),
        scratch_shapes=[
            pltpu.VMEM((H, S, D), jnp.bfloat16),
            pltpu.VMEM((H, S, D), jnp.bfloat16),
            pltpu.SemaphoreType.DMA((2,)),
            pltpu.SemaphoreType.DMA((2,)),
            pltpu.SemaphoreType.DMA((2,)),
            pltpu.SemaphoreType.DMA((2,)),
        ],
        compiler_params=pltpu.CompilerParams(collective_id=0),
    )(q, k, v)
    return jnp.transpose(out, (1, 0, 2))[None]


# device time: 105456 ns/iter; 2.0013x vs baseline; 2.0013x over previous
import jax
import jax.numpy as jnp
from jax import lax
from jax.experimental import pallas as pl
from jax.experimental.pallas import tpu as pltpu

H, S, D = 16, 1024, 128
HD = H * D
HALF = S // 2
NC = 16
CH = HALF // NC
QSCALE = (D ** -0.5) * 1.4426950408889634


def _body(q32_ref, k_ref, v_ref, out_ref, k_rem, v_rem, dn,
          sx, rx, sy, ry):
    my_x = lax.axis_index("x")
    my_y = lax.axis_index("y")
    r0 = HALF * my_y
    rB = HALF - r0

    barrier = pltpu.get_barrier_semaphore()
    for nbr in ((1 - my_x, my_y), (my_x, 1 - my_y)):
        pl.semaphore_signal(barrier, inc=1, device_id=nbr,
                            device_id_type=pl.DeviceIdType.MESH)
    pl.semaphore_wait(barrier, 2)

    p1 = {}
    for c in range(NC):
        for t, (src, dst) in enumerate(((k_ref, k_rem), (v_ref, v_rem))):
            rd = pltpu.make_async_remote_copy(
                src_ref=src.at[pl.ds(r0 + c * CH, CH)],
                dst_ref=dst.at[pl.ds(r0 + c * CH, CH)],
                send_sem=sx.at[t, c], recv_sem=rx.at[t, c],
                device_id=(1 - my_x, my_y),
                device_id_type=pl.DeviceIdType.MESH,
            )
            rd.start()
            p1[(t, c)] = rd

    fwd = {}

    def wait_and_forward(c):
        for t, buf in enumerate((k_rem, v_rem)):
            p1[(t, c)].wait_recv()
            rd = pltpu.make_async_remote_copy(
                src_ref=buf.at[pl.ds(r0 + c * CH, CH)],
                dst_ref=buf.at[pl.ds(r0 + c * CH, CH)],
                send_sem=sy.at[t, c], recv_sem=ry.at[t, c],
                device_id=(my_x, 1 - my_y),
                device_id_type=pl.DeviceIdType.MESH,
            )
            rd.start()
            fwd[(t, c)] = rd

    def qh(h):
        return (q32_ref[:, h * D:(h + 1) * D] * QSCALE).astype(jnp.bfloat16)

    def local_head(h):
        q = qh(h)
        s = lax.dot_general(q, k_ref[:, h * D:(h + 1) * D],
                            (((1,), (1,)), ((), ())),
                            preferred_element_type=jnp.float32)
        p = jnp.exp2(s)
        pb = p.astype(jnp.bfloat16)
        dn[:, h:h + 1] = jnp.sum(p, axis=1, keepdims=True)
        out_ref[:, h * D:(h + 1) * D] = lax.dot_general(
            pb, v_ref[:, h * D:(h + 1) * D], (((1,), (0,)), ((), ())),
            preferred_element_type=jnp.float32)

    def remote_block(h, row0, nrows, last):
        q = qh(h)
        s = lax.dot_general(q, k_rem[pl.ds(row0, nrows), pl.ds(h * D, D)],
                            (((1,), (1,)), ((), ())),
                            preferred_element_type=jnp.float32)
        p = jnp.exp2(s)
        pb = p.astype(jnp.bfloat16)
        acc = out_ref[:, h * D:(h + 1) * D] + lax.dot_general(
            pb, v_rem[pl.ds(row0, nrows), pl.ds(h * D, D)],
            (((1,), (0,)), ((), ())), preferred_element_type=jnp.float32)
        den = dn[:, h:h + 1] + jnp.sum(p, axis=1, keepdims=True)
        if last:
            out_ref[:, h * D:(h + 1) * D] = acc * (1.0 / den)
        else:
            out_ref[:, h * D:(h + 1) * D] = acc
            dn[:, h:h + 1] = den

    for h in range(8):
        local_head(h)
        wait_and_forward(h)
    for h in range(8):
        remote_block(h, r0, HALF // 2, last=False)
    for h in range(8, 12):
        local_head(h)
        wait_and_forward(h)
    for t in range(2):
        for c in range(8):
            fwd[(t, c)].wait_recv()
    for h in range(8):
        remote_block(h, rB, HALF // 2, last=False)
    for h in range(12, 16):
        local_head(h)
        wait_and_forward(h)
    for h in range(8, H):
        remote_block(h, r0, HALF // 2, last=False)
    for h in range(8, H):
        remote_block(h, rB, HALF // 2, last=False)
    for h in range(H):
        remote_block(h, r0 + HALF // 2, HALF // 2, last=False)
    for t in range(2):
        for c in range(8, NC):
            fwd[(t, c)].wait_recv()
    for h in range(H):
        remote_block(h, rB + HALF // 2, HALF // 2, last=True)

    for c in range(NC):
        for t in range(2):
            p1[(t, c)].wait_send()
            fwd[(t, c)].wait_send()


def kernel(Q, K, V):
    q32 = Q.reshape(S, HD)
    k = K.reshape(S, HD).astype(jnp.bfloat16)
    v = V.reshape(S, HD).astype(jnp.bfloat16)
    out = pl.pallas_call(
        _body,
        out_shape=jax.ShapeDtypeStruct((S, HD), jnp.float32),
        in_specs=[pl.BlockSpec(memory_space=pltpu.VMEM)] * 3,
        out_specs=pl.BlockSpec(memory_space=pltpu.VMEM),
        scratch_shapes=[
            pltpu.VMEM((S, HD), jnp.bfloat16),
            pltpu.VMEM((S, HD), jnp.bfloat16),
            pltpu.VMEM((S, H), jnp.float32),
            pltpu.SemaphoreType.DMA((2, NC)),
            pltpu.SemaphoreType.DMA((2, NC)),
            pltpu.SemaphoreType.DMA((2, NC)),
            pltpu.SemaphoreType.DMA((2, NC)),
        ],
        compiler_params=pltpu.CompilerParams(collective_id=0),
    )(q32, k, v)
    return out.reshape(1, S, H, D)
